# baseline (speedup 1.0000x reference)
import jax
import jax.numpy as jnp
from jax.experimental import pallas as pl
from jax.experimental.pallas import tpu as pltpu

L = 32
S = 64
D_L = 128
D_ATT = 128
LS = L * S


def _fused_kernel(x_ref, g_ref, wq_ref, wk_ref, vg_ref, wk1_ref, wk2_ref,
                  wra_ref, wrb_ref, wroot1_ref, qmask_ref, tmask_ref,
                  wgm_ref, wrel2_ref, wroot2_ref, spk_ref,
                  x2_ref, lw_ref, x1_ref, p1_ref, p2t_ref, gwt_ref):
    p = pl.program_id(0)
    j = pl.program_id(1)

    @pl.when(jnp.logical_and(p == 0, j == 0))
    def _prelude():
        g = g_ref[...]
        q = jnp.dot(g, wq_ref[...], preferred_element_type=jnp.float32)
        k = jnp.dot(g, wk_ref[...], preferred_element_type=jnp.float32)
        t = jnp.tanh(q[:, None, :] + k[None, :, :])
        s = jnp.sum(t * vg_ref[0][None, None, :], axis=-1)
        s = s - jnp.max(s, axis=-1, keepdims=True)
        e = jnp.exp(s)
        gw = e / jnp.sum(e, axis=-1, keepdims=True)
        gwt_ref[...] = gw.T
        x2d = x_ref[...]
        p1_ref[...] = jnp.tanh(
            jnp.dot(x2d, wk1_ref[...], preferred_element_type=jnp.float32))
        p2 = jnp.tanh(
            jnp.dot(x2d, wk2_ref[...], preferred_element_type=jnp.float32))
        p2t_ref[...] = jnp.transpose(p2.reshape(L, S, D_ATT), (0, 2, 1))

    @pl.when(p == 0)
    def _layer1():
        sc = jnp.dot(p1_ref[...], p2t_ref[j],
                     preferred_element_type=jnp.float32) * (
                         1.0 / jnp.sqrt(jnp.float32(D_ATT)))
        e = jnp.exp(sc) * tmask_ref[0]
        ssum = jnp.dot(e, jnp.ones((S, 1), jnp.float32),
                       preferred_element_type=jnp.float32)
        lw = e * (qmask_ref[...] / ssum)
        lw_ref[j] = lw
        msg = jax.lax.dot_general(
            lw.reshape(L, S, S), x_ref[...].reshape(L, S, D_L),
            (((2,), (1,)), ((0,), (0,))),
            preferred_element_type=jnp.float32)
        wg = wgm_ref[0] * gwt_ref[pl.ds(j, 1)]
        acc4 = jax.lax.dot_general(
            wg, msg, (((1,), (0,)), ((), ())),
            preferred_element_type=jnp.float32)
        bb = spk_ref[j] * 2

        def rel(a, d):
            wr = wra_ref if a == 0 else wrb_ref
            return wr[bb + d]

        agg = (jnp.dot(acc4[0], rel(0, 0),
                       preferred_element_type=jnp.float32)
               + jnp.dot(acc4[1], rel(0, 1),
                         preferred_element_type=jnp.float32)
               + jnp.dot(acc4[2], rel(1, 0),
                         preferred_element_type=jnp.float32)
               + jnp.dot(acc4[3], rel(1, 1),
                         preferred_element_type=jnp.float32))
        x1_ref[j] = jnp.dot(x_ref[pl.ds(j * S, S)], wroot1_ref[...],
                            preferred_element_type=jnp.float32) + agg

    @pl.when(p == 1)
    def _layer2():
        msg = jax.lax.dot_general(
            lw_ref[j].reshape(L, S, S), x1_ref[...],
            (((2,), (1,)), ((0,), (0,))),
            preferred_element_type=jnp.float32)
        msum = jnp.sum(msg, axis=0)
        x2_ref[0] = (jnp.dot(x1_ref[j], wroot2_ref[...],
                             preferred_element_type=jnp.float32)
                     + jnp.dot(msum, wrel2_ref[...],
                               preferred_element_type=jnp.float32))


@jax.jit
def kernel(global_features, local_features, speaker, length, Wq_g, Wk_g,
           v_g, Wk1_l, Wk2_l, W_rel1, W_root1, W_rel2, W_root2):
    f32 = jnp.float32
    x2d = local_features.astype(f32).reshape(LS, D_L)
    vg2 = v_g.reshape(1, D_ATT)
    spk = speaker.astype(jnp.int32)
    lng = length.astype(jnp.int32)
    wrel2 = W_rel2.reshape(D_L, D_L)
    sr = jnp.arange(S, dtype=jnp.int32)
    qmask = (sr[None, :] < lng[:, None]).astype(f32).reshape(LS, 1)
    tmask = (sr[None, :] < lng[:, None]).astype(f32).reshape(L, 1, S)
    spr = speaker.astype(f32)[None, None, :]
    ilt = (jnp.arange(L)[None, :, None] > jnp.arange(L)[None, None, :]
           ).astype(f32)
    wgm = jnp.concatenate(
        [(1.0 - spr) * ilt, (1.0 - spr) * (1.0 - ilt),
         spr * ilt, spr * (1.0 - ilt)], axis=0)
    wgm = jnp.transpose(wgm, (1, 0, 2))

    def full(arr):
        n = arr.ndim
        return pl.BlockSpec(arr.shape, lambda p, j, n=n: (0,) * n)

    x2 = pl.pallas_call(
        _fused_kernel,
        grid=(2, L),
        in_specs=[full(x2d), full(global_features), full(Wq_g), full(Wk_g),
                  full(vg2), full(Wk1_l), full(Wk2_l),
                  pl.BlockSpec((4, D_L, D_L), lambda p, j: (0, 0, 0)),
                  pl.BlockSpec((4, D_L, D_L), lambda p, j: (16, 0, 0)),
                  full(W_root1), full(qmask),
                  pl.BlockSpec((1, 1, S), lambda p, j: (j, 0, 0)),
                  pl.BlockSpec((1, 4, L), lambda p, j: (j, 0, 0)),
                  full(wrel2), full(W_root2),
                  pl.BlockSpec(memory_space=pltpu.SMEM)],
        out_specs=pl.BlockSpec((1, S, D_L), lambda p, j: (j, 0, 0)),
        out_shape=jax.ShapeDtypeStruct((L, S, D_L), f32),
        scratch_shapes=[pltpu.VMEM((L, LS, S), f32),
                        pltpu.VMEM((L, S, D_L), f32),
                        pltpu.VMEM((LS, D_ATT), f32),
                        pltpu.VMEM((L, D_ATT, S), f32),
                        pltpu.VMEM((L, L), f32)],
    )(x2d, global_features, Wq_g, Wk_g, vg2, Wk1_l, Wk2_l,
      W_rel1, W_rel1, W_root1, qmask, tmask, wgm, wrel2, W_root2, spk)

    return x2


# speedup vs baseline: 1.2182x; 1.2182x over previous
import jax
import jax.numpy as jnp
from jax.experimental import pallas as pl
from jax.experimental.pallas import tpu as pltpu

L = 32
S = 64
D_L = 128
D_ATT = 128
LS = L * S
JB = 4


def _fused_kernel(x_ref, g_ref, wq_ref, wk_ref, vg_ref, wk1_ref, wk2_ref,
                  wra_ref, wrb_ref, wroot1_ref, qmask_ref, tmask_ref,
                  wgm_ref, wrel2_ref, wroot2_ref, spk_ref,
                  x2_ref, lw_ref, x1_ref, p1_ref, p2t_ref, gwt_ref):
    p = pl.program_id(0)
    jj = pl.program_id(1)

    @pl.when(jnp.logical_and(p == 0, jj == 0))
    def _prelude():
        g = g_ref[...]
        q = jnp.dot(g, wq_ref[...], preferred_element_type=jnp.float32)
        k = jnp.dot(g, wk_ref[...], preferred_element_type=jnp.float32)
        t = jnp.tanh(q[:, None, :] + k[None, :, :])
        s = jnp.sum(t * vg_ref[0][None, None, :], axis=-1)
        s = s - jnp.max(s, axis=-1, keepdims=True)
        e = jnp.exp(s)
        gw = e / jnp.sum(e, axis=-1, keepdims=True)
        gwt_ref[...] = gw.T
        x2d = x_ref[...]
        p1_ref[...] = jnp.tanh(
            jnp.dot(x2d, wk1_ref[...], preferred_element_type=jnp.float32))
        p2 = jnp.tanh(
            jnp.dot(x2d, wk2_ref[...], preferred_element_type=jnp.float32))
        p2t_ref[...] = jnp.transpose(p2.reshape(L, S, D_ATT), (0, 2, 1))

    @pl.when(p == 0)
    def _layer1():
        for k in range(JB):
            j = jj * JB + k
            sc = jnp.dot(p1_ref[...], p2t_ref[j],
                         preferred_element_type=jnp.float32) * (
                             1.0 / jnp.sqrt(jnp.float32(D_ATT)))
            e = jnp.exp(sc) * tmask_ref[k]
            ssum = jnp.dot(e, jnp.ones((S, 1), jnp.float32),
                           preferred_element_type=jnp.float32)
            lw = e * (qmask_ref[...] / ssum)
            lw_ref[j] = lw
            msg = jax.lax.dot_general(
                lw.reshape(L, S, S), x_ref[...].reshape(L, S, D_L),
                (((2,), (1,)), ((0,), (0,))),
                preferred_element_type=jnp.float32)
            wg = wgm_ref[k] * gwt_ref[pl.ds(j, 1)]
            acc4 = jax.lax.dot_general(
                wg, msg, (((1,), (0,)), ((), ())),
                preferred_element_type=jnp.float32)
            bb = spk_ref[j] * 2

            def rel(a, d):
                wr = wra_ref if a == 0 else wrb_ref
                return wr[bb + d]

            agg = (jnp.dot(acc4[0], rel(0, 0),
                           preferred_element_type=jnp.float32)
                   + jnp.dot(acc4[1], rel(0, 1),
                             preferred_element_type=jnp.float32)
                   + jnp.dot(acc4[2], rel(1, 0),
                             preferred_element_type=jnp.float32)
                   + jnp.dot(acc4[3], rel(1, 1),
                             preferred_element_type=jnp.float32))
            x1_ref[j] = jnp.dot(x_ref[pl.ds(j * S, S)], wroot1_ref[...],
                                preferred_element_type=jnp.float32) + agg

    @pl.when(p == 1)
    def _layer2():
        for k in range(JB):
            j = jj * JB + k
            msg = jax.lax.dot_general(
                lw_ref[j].reshape(L, S, S), x1_ref[...],
                (((2,), (1,)), ((0,), (0,))),
                preferred_element_type=jnp.float32)
            msum = jnp.sum(msg, axis=0)
            x2_ref[k] = (jnp.dot(x1_ref[j], wroot2_ref[...],
                                 preferred_element_type=jnp.float32)
                         + jnp.dot(msum, wrel2_ref[...],
                                   preferred_element_type=jnp.float32))


@jax.jit
def kernel(global_features, local_features, speaker, length, Wq_g, Wk_g,
           v_g, Wk1_l, Wk2_l, W_rel1, W_root1, W_rel2, W_root2):
    f32 = jnp.float32
    x2d = local_features.astype(f32).reshape(LS, D_L)
    vg2 = v_g.reshape(1, D_ATT)
    spk = speaker.astype(jnp.int32)
    lng = length.astype(jnp.int32)
    wrel2 = W_rel2.reshape(D_L, D_L)
    sr = jnp.arange(S, dtype=jnp.int32)
    qmask = (sr[None, :] < lng[:, None]).astype(f32).reshape(LS, 1)
    tmask = (sr[None, :] < lng[:, None]).astype(f32).reshape(L, 1, S)
    spr = speaker.astype(f32)[None, None, :]
    ilt = (jnp.arange(L)[None, :, None] > jnp.arange(L)[None, None, :]
           ).astype(f32)
    wgm = jnp.concatenate(
        [(1.0 - spr) * ilt, (1.0 - spr) * (1.0 - ilt),
         spr * ilt, spr * (1.0 - ilt)], axis=0)
    wgm = jnp.transpose(wgm, (1, 0, 2))

    def full(arr):
        n = arr.ndim
        return pl.BlockSpec(arr.shape, lambda p, j, n=n: (0,) * n)

    x2 = pl.pallas_call(
        _fused_kernel,
        grid=(2, L // JB),
        in_specs=[full(x2d), full(global_features), full(Wq_g), full(Wk_g),
                  full(vg2), full(Wk1_l), full(Wk2_l),
                  pl.BlockSpec((4, D_L, D_L), lambda p, j: (0, 0, 0)),
                  pl.BlockSpec((4, D_L, D_L), lambda p, j: (16, 0, 0)),
                  full(W_root1), full(qmask),
                  pl.BlockSpec((JB, 1, S), lambda p, j: (j, 0, 0)),
                  pl.BlockSpec((JB, 4, L), lambda p, j: (j, 0, 0)),
                  full(wrel2), full(W_root2),
                  pl.BlockSpec(memory_space=pltpu.SMEM)],
        out_specs=pl.BlockSpec((JB, S, D_L), lambda p, j: (j, 0, 0)),
        out_shape=jax.ShapeDtypeStruct((L, S, D_L), f32),
        scratch_shapes=[pltpu.VMEM((L, LS, S), f32),
                        pltpu.VMEM((L, S, D_L), f32),
                        pltpu.VMEM((LS, D_ATT), f32),
                        pltpu.VMEM((L, D_ATT, S), f32),
                        pltpu.VMEM((L, L), f32)],
    )(x2d, global_features, Wq_g, Wk_g, vg2, Wk1_l, Wk2_l,
      W_rel1, W_rel1, W_root1, qmask, tmask, wgm, wrel2, W_root2, spk)

    return x2


# speedup vs baseline: 1.2595x; 1.0339x over previous
import jax
import jax.numpy as jnp
from jax.experimental import pallas as pl
from jax.experimental.pallas import tpu as pltpu

L = 32
S = 64
D_L = 128
D_ATT = 128
LS = L * S
JB = 8


def _fused_kernel(x_ref, g_ref, wq_ref, wk_ref, vg_ref, wk1_ref, wk2_ref,
                  wra_ref, wrb_ref, wroot1_ref, qmask_ref, tmask_ref,
                  wgm_ref, wrel2_ref, wroot2_ref, spk_ref,
                  x2_ref, lw_ref, x1_ref, p1_ref, p2t_ref, gwt_ref):
    p = pl.program_id(0)
    jj = pl.program_id(1)

    @pl.when(jnp.logical_and(p == 0, jj == 0))
    def _prelude():
        g = g_ref[...]
        q = jnp.dot(g, wq_ref[...], preferred_element_type=jnp.float32)
        k = jnp.dot(g, wk_ref[...], preferred_element_type=jnp.float32)
        t = jnp.tanh(q[:, None, :] + k[None, :, :])
        s = jnp.sum(t * vg_ref[0][None, None, :], axis=-1)
        s = s - jnp.max(s, axis=-1, keepdims=True)
        e = jnp.exp(s)
        gw = e / jnp.sum(e, axis=-1, keepdims=True)
        gwt_ref[...] = gw.T
        x2d = x_ref[...]
        p1_ref[...] = jnp.tanh(
            jnp.dot(x2d, wk1_ref[...], preferred_element_type=jnp.float32))
        p2 = jnp.tanh(
            jnp.dot(x2d, wk2_ref[...], preferred_element_type=jnp.float32))
        p2t_ref[...] = jnp.transpose(p2.reshape(L, S, D_ATT), (0, 2, 1))

    @pl.when(p == 0)
    def _layer1():
        for k in range(JB):
            j = jj * JB + k
            sc = jnp.dot(p1_ref[...], p2t_ref[j],
                         preferred_element_type=jnp.float32) * (
                             1.0 / jnp.sqrt(jnp.float32(D_ATT)))
            e = jnp.exp(sc) * tmask_ref[k]
            ssum = jnp.dot(e, jnp.ones((S, 1), jnp.float32),
                           preferred_element_type=jnp.float32)
            lw = e * (qmask_ref[...] / ssum)
            lw_ref[j] = lw
            msg = jax.lax.dot_general(
                lw.reshape(L, S, S), x_ref[...].reshape(L, S, D_L),
                (((2,), (1,)), ((0,), (0,))),
                preferred_element_type=jnp.float32)
            wg = wgm_ref[k] * gwt_ref[pl.ds(j, 1)]
            acc4 = jax.lax.dot_general(
                wg, msg, (((1,), (0,)), ((), ())),
                preferred_element_type=jnp.float32)
            bb = spk_ref[j] * 2

            def rel(a, d):
                wr = wra_ref if a == 0 else wrb_ref
                return wr[bb + d]

            agg = (jnp.dot(acc4[0], rel(0, 0),
                           preferred_element_type=jnp.float32)
                   + jnp.dot(acc4[1], rel(0, 1),
                             preferred_element_type=jnp.float32)
                   + jnp.dot(acc4[2], rel(1, 0),
                             preferred_element_type=jnp.float32)
                   + jnp.dot(acc4[3], rel(1, 1),
                             preferred_element_type=jnp.float32))
            x1_ref[j] = jnp.dot(x_ref[pl.ds(j * S, S)], wroot1_ref[...],
                                preferred_element_type=jnp.float32) + agg

    @pl.when(p == 1)
    def _layer2():
        for k in range(JB):
            j = jj * JB + k
            msg = jax.lax.dot_general(
                lw_ref[j].reshape(L, S, S), x1_ref[...],
                (((2,), (1,)), ((0,), (0,))),
                preferred_element_type=jnp.float32)
            msum = jnp.sum(msg, axis=0)
            x2_ref[k] = (jnp.dot(x1_ref[j], wroot2_ref[...],
                                 preferred_element_type=jnp.float32)
                         + jnp.dot(msum, wrel2_ref[...],
                                   preferred_element_type=jnp.float32))


@jax.jit
def kernel(global_features, local_features, speaker, length, Wq_g, Wk_g,
           v_g, Wk1_l, Wk2_l, W_rel1, W_root1, W_rel2, W_root2):
    f32 = jnp.float32
    x2d = local_features.astype(f32).reshape(LS, D_L)
    vg2 = v_g.reshape(1, D_ATT)
    spk = speaker.astype(jnp.int32)
    lng = length.astype(jnp.int32)
    wrel2 = W_rel2.reshape(D_L, D_L)
    sr = jnp.arange(S, dtype=jnp.int32)
    qmask = (sr[None, :] < lng[:, None]).astype(f32).reshape(LS, 1)
    tmask = (sr[None, :] < lng[:, None]).astype(f32).reshape(L, 1, S)
    spr = speaker.astype(f32)[None, None, :]
    ilt = (jnp.arange(L)[None, :, None] > jnp.arange(L)[None, None, :]
           ).astype(f32)
    wgm = jnp.concatenate(
        [(1.0 - spr) * ilt, (1.0 - spr) * (1.0 - ilt),
         spr * ilt, spr * (1.0 - ilt)], axis=0)
    wgm = jnp.transpose(wgm, (1, 0, 2))

    def full(arr):
        n = arr.ndim
        return pl.BlockSpec(arr.shape, lambda p, j, n=n: (0,) * n)

    x2 = pl.pallas_call(
        _fused_kernel,
        grid=(2, L // JB),
        in_specs=[full(x2d), full(global_features), full(Wq_g), full(Wk_g),
                  full(vg2), full(Wk1_l), full(Wk2_l),
                  pl.BlockSpec((4, D_L, D_L), lambda p, j: (0, 0, 0)),
                  pl.BlockSpec((4, D_L, D_L), lambda p, j: (16, 0, 0)),
                  full(W_root1), full(qmask),
                  pl.BlockSpec((JB, 1, S), lambda p, j: (j, 0, 0)),
                  pl.BlockSpec((JB, 4, L), lambda p, j: (j, 0, 0)),
                  full(wrel2), full(W_root2),
                  pl.BlockSpec(memory_space=pltpu.SMEM)],
        out_specs=pl.BlockSpec((JB, S, D_L), lambda p, j: (j, 0, 0)),
        out_shape=jax.ShapeDtypeStruct((L, S, D_L), f32),
        scratch_shapes=[pltpu.VMEM((L, LS, S), f32),
                        pltpu.VMEM((L, S, D_L), f32),
                        pltpu.VMEM((LS, D_ATT), f32),
                        pltpu.VMEM((L, D_ATT, S), f32),
                        pltpu.VMEM((L, L), f32)],
    )(x2d, global_features, Wq_g, Wk_g, vg2, Wk1_l, Wk2_l,
      W_rel1, W_rel1, W_root1, qmask, tmask, wgm, wrel2, W_root2, spk)

    return x2


# speedup vs baseline: 1.2742x; 1.0117x over previous
import jax
import jax.numpy as jnp
from jax.experimental import pallas as pl
from jax.experimental.pallas import tpu as pltpu

L = 32
S = 64
D_L = 128
D_ATT = 128
LS = L * S
JB = 16


def _fused_kernel(x_ref, g_ref, wq_ref, wk_ref, vg_ref, wk1_ref, wk2_ref,
                  wra_ref, wrb_ref, wroot1_ref, qmask_ref, tmask_ref,
                  wgm_ref, wrel2_ref, wroot2_ref, spk_ref,
                  x2_ref, lw_ref, x1_ref, p1_ref, p2t_ref, gwt_ref):
    p = pl.program_id(0)
    jj = pl.program_id(1)

    @pl.when(jnp.logical_and(p == 0, jj == 0))
    def _prelude():
        g = g_ref[...]
        q = jnp.dot(g, wq_ref[...], preferred_element_type=jnp.float32)
        k = jnp.dot(g, wk_ref[...], preferred_element_type=jnp.float32)
        t = jnp.tanh(q[:, None, :] + k[None, :, :])
        s = jnp.sum(t * vg_ref[0][None, None, :], axis=-1)
        s = s - jnp.max(s, axis=-1, keepdims=True)
        e = jnp.exp(s)
        gw = e / jnp.sum(e, axis=-1, keepdims=True)
        gwt_ref[...] = gw.T
        x2d = x_ref[...]
        p1_ref[...] = jnp.tanh(
            jnp.dot(x2d, wk1_ref[...], preferred_element_type=jnp.float32))
        p2 = jnp.tanh(
            jnp.dot(x2d, wk2_ref[...], preferred_element_type=jnp.float32))
        p2t_ref[...] = jnp.transpose(p2.reshape(L, S, D_ATT), (0, 2, 1))

    @pl.when(p == 0)
    def _layer1():
        for k in range(JB):
            j = jj * JB + k
            sc = jnp.dot(p1_ref[...], p2t_ref[j],
                         preferred_element_type=jnp.float32) * (
                             1.0 / jnp.sqrt(jnp.float32(D_ATT)))
            e = jnp.exp(sc) * tmask_ref[k]
            ssum = jnp.dot(e, jnp.ones((S, 1), jnp.float32),
                           preferred_element_type=jnp.float32)
            lw = e * (qmask_ref[...] / ssum)
            lw_ref[j] = lw
            msg = jax.lax.dot_general(
                lw.reshape(L, S, S), x_ref[...].reshape(L, S, D_L),
                (((2,), (1,)), ((0,), (0,))),
                preferred_element_type=jnp.float32)
            wg = wgm_ref[k] * gwt_ref[pl.ds(j, 1)]
            acc4 = jax.lax.dot_general(
                wg, msg, (((1,), (0,)), ((), ())),
                preferred_element_type=jnp.float32)
            bb = spk_ref[j] * 2

            def rel(a, d):
                wr = wra_ref if a == 0 else wrb_ref
                return wr[bb + d]

            agg = (jnp.dot(acc4[0], rel(0, 0),
                           preferred_element_type=jnp.float32)
                   + jnp.dot(acc4[1], rel(0, 1),
                             preferred_element_type=jnp.float32)
                   + jnp.dot(acc4[2], rel(1, 0),
                             preferred_element_type=jnp.float32)
                   + jnp.dot(acc4[3], rel(1, 1),
                             preferred_element_type=jnp.float32))
            x1_ref[j] = jnp.dot(x_ref[pl.ds(j * S, S)], wroot1_ref[...],
                                preferred_element_type=jnp.float32) + agg

    @pl.when(p == 1)
    def _layer2():
        for k in range(JB):
            j = jj * JB + k
            msg = jax.lax.dot_general(
                lw_ref[j].reshape(L, S, S), x1_ref[...],
                (((2,), (1,)), ((0,), (0,))),
                preferred_element_type=jnp.float32)
            msum = jnp.sum(msg, axis=0)
            x2_ref[k] = (jnp.dot(x1_ref[j], wroot2_ref[...],
                                 preferred_element_type=jnp.float32)
                         + jnp.dot(msum, wrel2_ref[...],
                                   preferred_element_type=jnp.float32))


@jax.jit
def kernel(global_features, local_features, speaker, length, Wq_g, Wk_g,
           v_g, Wk1_l, Wk2_l, W_rel1, W_root1, W_rel2, W_root2):
    f32 = jnp.float32
    x2d = local_features.astype(f32).reshape(LS, D_L)
    vg2 = v_g.reshape(1, D_ATT)
    spk = speaker.astype(jnp.int32)
    lng = length.astype(jnp.int32)
    wrel2 = W_rel2.reshape(D_L, D_L)
    sr = jnp.arange(S, dtype=jnp.int32)
    qmask = (sr[None, :] < lng[:, None]).astype(f32).reshape(LS, 1)
    tmask = (sr[None, :] < lng[:, None]).astype(f32).reshape(L, 1, S)
    spr = speaker.astype(f32)[None, None, :]
    ilt = (jnp.arange(L)[None, :, None] > jnp.arange(L)[None, None, :]
           ).astype(f32)
    wgm = jnp.concatenate(
        [(1.0 - spr) * ilt, (1.0 - spr) * (1.0 - ilt),
         spr * ilt, spr * (1.0 - ilt)], axis=0)
    wgm = jnp.transpose(wgm, (1, 0, 2))

    def full(arr):
        n = arr.ndim
        return pl.BlockSpec(arr.shape, lambda p, j, n=n: (0,) * n)

    x2 = pl.pallas_call(
        _fused_kernel,
        grid=(2, L // JB),
        in_specs=[full(x2d), full(global_features), full(Wq_g), full(Wk_g),
                  full(vg2), full(Wk1_l), full(Wk2_l),
                  pl.BlockSpec((4, D_L, D_L), lambda p, j: (0, 0, 0)),
                  pl.BlockSpec((4, D_L, D_L), lambda p, j: (16, 0, 0)),
                  full(W_root1), full(qmask),
                  pl.BlockSpec((JB, 1, S), lambda p, j: (j, 0, 0)),
                  pl.BlockSpec((JB, 4, L), lambda p, j: (j, 0, 0)),
                  full(wrel2), full(W_root2),
                  pl.BlockSpec(memory_space=pltpu.SMEM)],
        out_specs=pl.BlockSpec((JB, S, D_L), lambda p, j: (j, 0, 0)),
        out_shape=jax.ShapeDtypeStruct((L, S, D_L), f32),
        scratch_shapes=[pltpu.VMEM((L, LS, S), f32),
                        pltpu.VMEM((L, S, D_L), f32),
                        pltpu.VMEM((LS, D_ATT), f32),
                        pltpu.VMEM((L, D_ATT, S), f32),
                        pltpu.VMEM((L, L), f32)],
    )(x2d, global_features, Wq_g, Wk_g, vg2, Wk1_l, Wk2_l,
      W_rel1, W_rel1, W_root1, qmask, tmask, wgm, wrel2, W_root2, spk)

    return x2
